# baseline (speedup 1.0000x reference)
import functools

import jax
import jax.numpy as jnp
from jax.experimental import pallas as pl
from jax.experimental.pallas import tpu as pltpu

_ALPHA = 0.2
_EPS = 1e-5


def _hst_body(x_ref, w_ref, b_ref, asrc_ref, adst_ref,
              h_ref, p_ref, pa_ref, q_ref, qa_ref, *, n, tm2):
    i = pl.program_id(0)
    h = jax.lax.dot_general(
        x_ref[...], w_ref[...], (((1,), (1,)), ((), ())),
        preferred_element_type=jnp.float32) + b_ref[...]
    row = i * tm2 + jax.lax.broadcasted_iota(jnp.int32, (tm2, 1), 0)
    h = jnp.where(row < n, h, 0.0)
    h_ref[...] = h.astype(jnp.bfloat16)
    s = jax.lax.dot_general(
        h, asrc_ref[...], (((1,), (0,)), ((), ())),
        preferred_element_type=jnp.float32)
    t = jax.lax.dot_general(
        h, adst_ref[...], (((1,), (0,)), ((), ())),
        preferred_element_type=jnp.float32)
    p_ref[...] = jnp.exp(-s)
    pa_ref[...] = jnp.exp(-_ALPHA * s)
    q_ref[...] = jnp.exp(-t)
    qa_ref[...] = jnp.exp(-_ALPHA * t)


def _gat_body(adj_ref, p_ref, pa_ref, q_ref, qa_ref, h_ref, g_ref, be_ref,
              o_ref, wa_ref, wb_ref, *, n, tm, tk, nk, rc):
    tail = n - (nk - 1) * tk
    acc = jnp.zeros((tm, h_ref.shape[1]), jnp.float32)
    for k in range(nk):
        w_ref = wa_ref if k % 2 == 0 else wb_ref
        q = q_ref[:, pl.ds(k * tk, tk)]
        qa = qa_ref[:, pl.ds(k * tk, tk)]
        for c in range(tm // rc):
            sl = pl.ds(c * rc, rc)
            m1 = p_ref[sl, :] * q
            m2 = pa_ref[sl, :] * qa
            w = adj_ref[sl, pl.ds(k * tk, tk)] * jnp.minimum(m1, m2)
            w_ref[sl, :] = w.astype(jnp.bfloat16)
        if k == nk - 1 and tail < tk:
            w_ref[:, pl.ds(tail, tk - tail)] = jnp.zeros(
                (tm, tk - tail), jnp.bfloat16)
        acc = acc + jax.lax.dot_general(
            w_ref[...], h_ref[pl.ds(k * tk, tk), :], (((1,), (0,)), ((), ())),
            preferred_element_type=jnp.float32)

    mean = jnp.mean(acc, axis=1, keepdims=True)
    cen = acc - mean
    var = jnp.mean(cen * cen, axis=1, keepdims=True)
    hn = cen * jax.lax.rsqrt(var + _EPS) * g_ref[...] + be_ref[...]
    o_ref[...] = jnp.where(hn > 0, hn, jnp.exp(jnp.minimum(hn, 0.0)) - 1.0)


def kernel(input, adj, W, b, a, gamma, beta):
    n, f = input.shape
    tk = 2048
    nk = pl.cdiv(n, tk)
    npad = nk * tk

    tm2 = npad // 5 if npad % 5 == 0 else npad
    asrc = a[0, :f].reshape(f, 1)
    adst = a[0, f:].reshape(f, 1)
    col = jax.ShapeDtypeStruct((npad, 1), jnp.float32)
    h, p, pa, q, qa = pl.pallas_call(
        functools.partial(_hst_body, n=n, tm2=tm2),
        grid=(npad // tm2,),
        in_specs=[
            pl.BlockSpec((tm2, f), lambda i: (i, 0)),
            pl.BlockSpec((f, f), lambda i: (0, 0)),
            pl.BlockSpec((1, f), lambda i: (0, 0)),
            pl.BlockSpec((f, 1), lambda i: (0, 0)),
            pl.BlockSpec((f, 1), lambda i: (0, 0)),
        ],
        out_specs=[
            pl.BlockSpec((tm2, f), lambda i: (i, 0)),
            pl.BlockSpec((tm2, 1), lambda i: (i, 0)),
            pl.BlockSpec((tm2, 1), lambda i: (i, 0)),
            pl.BlockSpec((tm2, 1), lambda i: (i, 0)),
            pl.BlockSpec((tm2, 1), lambda i: (i, 0)),
        ],
        out_shape=[jax.ShapeDtypeStruct((npad, f), jnp.bfloat16),
                   col, col, col, col],
        compiler_params=pltpu.CompilerParams(
            dimension_semantics=("parallel",)),
    )(input, W, b.reshape(1, f), asrc, adst)

    q_row = q.reshape(1, npad)
    qa_row = qa.reshape(1, npad)

    tm = 200 if n % 200 == 0 else (128 if n % 128 == 0 else 8)
    rc = 16 if tm % 16 == 0 else 8

    out = pl.pallas_call(
        functools.partial(_gat_body, n=n, tm=tm, tk=tk, nk=nk, rc=rc),
        grid=(n // tm,),
        in_specs=[
            pl.BlockSpec((tm, npad), lambda i: (i, 0)),
            pl.BlockSpec((tm, 1), lambda i: (i, 0)),
            pl.BlockSpec((tm, 1), lambda i: (i, 0)),
            pl.BlockSpec((1, npad), lambda i: (0, 0)),
            pl.BlockSpec((1, npad), lambda i: (0, 0)),
            pl.BlockSpec((npad, f), lambda i: (0, 0)),
            pl.BlockSpec((1, f), lambda i: (0, 0)),
            pl.BlockSpec((1, f), lambda i: (0, 0)),
        ],
        out_specs=pl.BlockSpec((tm, f), lambda i: (i, 0)),
        out_shape=jax.ShapeDtypeStruct((n, f), jnp.float32),
        scratch_shapes=[
            pltpu.VMEM((tm, tk), jnp.bfloat16),
            pltpu.VMEM((tm, tk), jnp.bfloat16),
        ],
        compiler_params=pltpu.CompilerParams(
            dimension_semantics=("parallel",),
            vmem_limit_bytes=64 * 1024 * 1024),
    )(adj, p, pa, q_row, qa_row, h,
      gamma.reshape(1, f), beta.reshape(1, f))
    return out


# speedup vs baseline: 1.1310x; 1.1310x over previous
import functools

import jax
import jax.numpy as jnp
from jax.experimental import pallas as pl
from jax.experimental.pallas import tpu as pltpu

_ALPHA = 0.2
_EPS = 1e-5


def _hst_body(x_ref, w_ref, b_ref, asrc_ref, adst_ref,
              h_ref, p_ref, pa_ref, q_ref, qa_ref, *, n, tm2):
    i = pl.program_id(0)
    h = jax.lax.dot_general(
        x_ref[...], w_ref[...], (((1,), (1,)), ((), ())),
        preferred_element_type=jnp.float32) + b_ref[...]
    row = i * tm2 + jax.lax.broadcasted_iota(jnp.int32, (tm2, 1), 0)
    h = jnp.where(row < n, h, 0.0)
    h_ref[...] = h.astype(jnp.bfloat16)
    s = jax.lax.dot_general(
        h, asrc_ref[...], (((1,), (0,)), ((), ())),
        preferred_element_type=jnp.float32)
    t = jax.lax.dot_general(
        h, adst_ref[...], (((1,), (0,)), ((), ())),
        preferred_element_type=jnp.float32)
    p_ref[...] = jnp.exp(-s)
    pa_ref[...] = jnp.exp(-_ALPHA * s)
    q_ref[...] = jnp.exp(-t)
    qa_ref[...] = jnp.exp(-_ALPHA * t)


def _gat_body(adj_ref, p_ref, pa_ref, q_ref, qa_ref, h_ref, g_ref, be_ref,
              o_ref, wa_ref, wb_ref, *, n, tm, tk, nk, rc):
    tail = n - (nk - 1) * tk
    acc = jnp.zeros((tm, h_ref.shape[1]), jnp.float32)
    for k in range(nk):
        w_ref = wa_ref if k % 2 == 0 else wb_ref
        q = q_ref[:, pl.ds(k * tk, tk)]
        qa = qa_ref[:, pl.ds(k * tk, tk)]
        for c in range(tm // rc):
            sl = pl.ds(c * rc, rc)
            m1 = p_ref[sl, :] * q
            m2 = pa_ref[sl, :] * qa
            w = adj_ref[sl, pl.ds(k * tk, tk)] * jnp.minimum(m1, m2)
            w_ref[sl, :] = w.astype(jnp.bfloat16)
        if k == nk - 1 and tail < tk:
            w_ref[:, pl.ds(tail, tk - tail)] = jnp.zeros(
                (tm, tk - tail), jnp.bfloat16)
        acc = acc + jax.lax.dot_general(
            w_ref[...], h_ref[pl.ds(k * tk, tk), :], (((1,), (0,)), ((), ())),
            preferred_element_type=jnp.float32)

    mean = jnp.mean(acc, axis=1, keepdims=True)
    cen = acc - mean
    var = jnp.mean(cen * cen, axis=1, keepdims=True)
    hn = cen * jax.lax.rsqrt(var + _EPS) * g_ref[...] + be_ref[...]
    o_ref[...] = jnp.where(hn > 0, hn, jnp.exp(jnp.minimum(hn, 0.0)) - 1.0)


def kernel(input, adj, W, b, a, gamma, beta):
    n, f = input.shape
    tk = 2048
    nk = pl.cdiv(n, tk)
    npad = nk * tk

    tm2 = npad // 5 if npad % 5 == 0 else npad
    asrc = a[0, :f].reshape(f, 1)
    adst = a[0, f:].reshape(f, 1)
    col = jax.ShapeDtypeStruct((npad, 1), jnp.float32)
    h, p, pa, q, qa = pl.pallas_call(
        functools.partial(_hst_body, n=n, tm2=tm2),
        grid=(npad // tm2,),
        in_specs=[
            pl.BlockSpec((tm2, f), lambda i: (i, 0)),
            pl.BlockSpec((f, f), lambda i: (0, 0)),
            pl.BlockSpec((1, f), lambda i: (0, 0)),
            pl.BlockSpec((f, 1), lambda i: (0, 0)),
            pl.BlockSpec((f, 1), lambda i: (0, 0)),
        ],
        out_specs=[
            pl.BlockSpec((tm2, f), lambda i: (i, 0)),
            pl.BlockSpec((tm2, 1), lambda i: (i, 0)),
            pl.BlockSpec((tm2, 1), lambda i: (i, 0)),
            pl.BlockSpec((tm2, 1), lambda i: (i, 0)),
            pl.BlockSpec((tm2, 1), lambda i: (i, 0)),
        ],
        out_shape=[jax.ShapeDtypeStruct((npad, f), jnp.bfloat16),
                   col, col, col, col],
        compiler_params=pltpu.CompilerParams(
            dimension_semantics=("parallel",)),
    )(input, W, b.reshape(1, f), asrc, adst)

    q_row = q.reshape(1, npad)
    qa_row = qa.reshape(1, npad)

    tm = 400 if n % 400 == 0 else (128 if n % 128 == 0 else 8)
    rc = 16 if tm % 16 == 0 else 8

    out = pl.pallas_call(
        functools.partial(_gat_body, n=n, tm=tm, tk=tk, nk=nk, rc=rc),
        grid=(n // tm,),
        in_specs=[
            pl.BlockSpec((tm, npad), lambda i: (i, 0)),
            pl.BlockSpec((tm, 1), lambda i: (i, 0)),
            pl.BlockSpec((tm, 1), lambda i: (i, 0)),
            pl.BlockSpec((1, npad), lambda i: (0, 0)),
            pl.BlockSpec((1, npad), lambda i: (0, 0)),
            pl.BlockSpec((npad, f), lambda i: (0, 0)),
            pl.BlockSpec((1, f), lambda i: (0, 0)),
            pl.BlockSpec((1, f), lambda i: (0, 0)),
        ],
        out_specs=pl.BlockSpec((tm, f), lambda i: (i, 0)),
        out_shape=jax.ShapeDtypeStruct((n, f), jnp.float32),
        scratch_shapes=[
            pltpu.VMEM((tm, tk), jnp.bfloat16),
            pltpu.VMEM((tm, tk), jnp.bfloat16),
        ],
        compiler_params=pltpu.CompilerParams(
            dimension_semantics=("parallel",),
            vmem_limit_bytes=64 * 1024 * 1024),
    )(adj, p, pa, q_row, qa_row, h,
      gamma.reshape(1, f), beta.reshape(1, f))
    return out


# speedup vs baseline: 1.2710x; 1.1238x over previous
import functools

import jax
import jax.numpy as jnp
from jax.experimental import pallas as pl
from jax.experimental.pallas import tpu as pltpu

_ALPHA = 0.2
_EPS = 1e-5


def _gat_body(adj_ref, x_ref, wt_ref, b_ref, asrc_ref, adst_ref,
              g_ref, be_ref, o_ref,
              h_ref, p_ref, pa_ref, q_ref, qa_ref, wa_ref, wb_ref,
              *, n, tm, tk, nk, rc, sb):
    npad = nk * tk
    f = x_ref.shape[1]
    tail = n - (nk - 1) * tk

    @pl.when(pl.program_id(0) == 0)
    def _():
        for s0 in range(n // sb):
            sl = pl.ds(s0 * sb, sb)
            hv = jax.lax.dot_general(
                x_ref[sl, :], wt_ref[...], (((1,), (1,)), ((), ())),
                preferred_element_type=jnp.float32) + b_ref[...]
            h_ref[sl, :] = hv.astype(jnp.bfloat16)
            s = jax.lax.dot_general(
                hv, asrc_ref[...], (((1,), (0,)), ((), ())),
                preferred_element_type=jnp.float32)
            p_ref[sl, :] = jnp.exp(-s)
            pa_ref[sl, :] = jnp.exp(-_ALPHA * s)
            t = jax.lax.dot_general(
                adst_ref[...], hv, (((1,), (1,)), ((), ())),
                preferred_element_type=jnp.float32)
            q_ref[:, sl] = jnp.exp(-t)
            qa_ref[:, sl] = jnp.exp(-_ALPHA * t)
        if npad > n:
            h_ref[pl.ds(n, npad - n), :] = jnp.zeros((npad - n, f),
                                                     jnp.bfloat16)
            q_ref[:, pl.ds(n, npad - n)] = jnp.zeros((1, npad - n),
                                                     jnp.float32)
            qa_ref[:, pl.ds(n, npad - n)] = jnp.zeros((1, npad - n),
                                                      jnp.float32)

    row0 = pl.program_id(0) * tm
    acc = jnp.zeros((tm, f), jnp.float32)
    for k in range(nk):
        w_ref = wa_ref if k % 2 == 0 else wb_ref
        q = q_ref[:, pl.ds(k * tk, tk)]
        qa = qa_ref[:, pl.ds(k * tk, tk)]
        for c in range(tm // rc):
            sl = pl.ds(c * rc, rc)
            m1 = p_ref[pl.ds(row0 + c * rc, rc), :] * q
            m2 = pa_ref[pl.ds(row0 + c * rc, rc), :] * qa
            w = adj_ref[sl, pl.ds(k * tk, tk)] * jnp.minimum(m1, m2)
            w_ref[sl, :] = w.astype(jnp.bfloat16)
        if k == nk - 1 and tail < tk:
            w_ref[:, pl.ds(tail, tk - tail)] = jnp.zeros(
                (tm, tk - tail), jnp.bfloat16)
        acc = acc + jax.lax.dot_general(
            w_ref[...], h_ref[pl.ds(k * tk, tk), :], (((1,), (0,)), ((), ())),
            preferred_element_type=jnp.float32)

    mean = jnp.mean(acc, axis=1, keepdims=True)
    cen = acc - mean
    var = jnp.mean(cen * cen, axis=1, keepdims=True)
    hn = cen * jax.lax.rsqrt(var + _EPS) * g_ref[...] + be_ref[...]
    o_ref[...] = jnp.where(hn > 0, hn, jnp.exp(jnp.minimum(hn, 0.0)) - 1.0)


def kernel(input, adj, W, b, a, gamma, beta):
    n, f = input.shape
    tk = 2048
    nk = pl.cdiv(n, tk)
    npad = nk * tk
    tm = 400 if n % 400 == 0 else (128 if n % 128 == 0 else 8)
    rc = 16 if tm % 16 == 0 else 8
    sb = 1000 if n % 1000 == 0 else tm

    full = lambda shape: pl.BlockSpec(shape, lambda i: tuple(0 for _ in shape))
    out = pl.pallas_call(
        functools.partial(_gat_body, n=n, tm=tm, tk=tk, nk=nk, rc=rc, sb=sb),
        grid=(n // tm,),
        in_specs=[
            pl.BlockSpec((tm, npad), lambda i: (i, 0)),
            full((n, f)),
            full((f, f)),
            full((1, f)),
            full((f, 1)),
            full((1, f)),
            full((1, f)),
            full((1, f)),
        ],
        out_specs=pl.BlockSpec((tm, f), lambda i: (i, 0)),
        out_shape=jax.ShapeDtypeStruct((n, f), jnp.float32),
        scratch_shapes=[
            pltpu.VMEM((npad, f), jnp.bfloat16),
            pltpu.VMEM((npad, 1), jnp.float32),
            pltpu.VMEM((npad, 1), jnp.float32),
            pltpu.VMEM((1, npad), jnp.float32),
            pltpu.VMEM((1, npad), jnp.float32),
            pltpu.VMEM((tm, tk), jnp.bfloat16),
            pltpu.VMEM((tm, tk), jnp.bfloat16),
        ],
        compiler_params=pltpu.CompilerParams(
            dimension_semantics=("arbitrary",),
            vmem_limit_bytes=64 * 1024 * 1024),
    )(adj, input, W, b.reshape(1, f), a[0, :f].reshape(f, 1),
      a[0, f:].reshape(1, f), gamma.reshape(1, f), beta.reshape(1, f))
    return out
